# Initial kernel scaffold; baseline (speedup 1.0000x reference)
import jax
import jax.numpy as jnp
from jax.experimental import pallas as pl


def kernel(X, table, W, bias):
    raise NotImplementedError("write your pallas kernel here")


# speedup vs baseline: 1.0262x; 1.0262x over previous
import functools

import jax
import jax.numpy as jnp
from jax import lax
from jax.experimental import pallas as pl
from jax.experimental.pallas import tpu as pltpu
from jax.experimental.pallas import tpu_sc as plsc

B = 4096
N_SPARSE = 26
N_DENSE = 13
N_COLS = N_SPARSE + N_DENSE
VOCAB = 100000


@functools.cache
def _build():
    info = plsc.get_sparse_core_info()
    NC, NS, L = info.num_cores, info.num_subcores, info.num_lanes
    NW = NC * NS
    RPW = B // NW
    NG = RPW // L
    mesh = plsc.VectorSubcoreMesh(core_axis_name="c", subcore_axis_name="s")

    @functools.partial(
        pl.kernel,
        mesh=mesh,
        compiler_params=pltpu.CompilerParams(needs_layout_passes=False),
        out_type=jax.ShapeDtypeStruct((B,), jnp.float32),
        scratch_types=[
            pltpu.VMEM((RPW * N_COLS,), jnp.float32),
            pltpu.VMEM((N_SPARSE, RPW), jnp.int32),
            pltpu.VMEM((N_SPARSE, RPW), jnp.float32),
            pltpu.VMEM((RPW,), jnp.float32),
            pltpu.VMEM((L,), jnp.float32),
            pltpu.SemaphoreType.DMA,
        ],
    )
    def k(x_hbm, table_hbm, wb_hbm, out_hbm, x_v, idx_v, emb_v, acc_v, w_v, sem):
        wid = lax.axis_index("s") * NC + lax.axis_index("c")
        base = wid * RPW
        pltpu.sync_copy(x_hbm.at[pl.ds(base * N_COLS, RPW * N_COLS)], x_v)
        pltpu.sync_copy(wb_hbm, w_v)
        lanes = lax.iota(jnp.int32, L)

        for g in range(NG):
            flat = (lanes + (g * L)) * N_COLS
            for j in range(N_SPARSE):
                xf = plsc.load_gather(x_v, [flat + j])
                idx_v[j, pl.ds(g * L, L)] = xf.astype(jnp.int32) + j * VOCAB

        copies = [
            pltpu.async_copy(table_hbm.at[idx_v.at[j]], emb_v.at[j], sem)
            for j in range(N_SPARSE)
        ]

        wv = w_v[...]
        for g in range(NG):
            flat = (lanes + (g * L)) * N_COLS
            acc = jnp.full((L,), wv[N_DENSE], jnp.float32)
            for d in range(N_DENSE):
                xf = plsc.load_gather(x_v, [flat + N_SPARSE + d])
                acc = acc + xf * wv[d]
            acc_v[pl.ds(g * L, L)] = acc

        for c in copies:
            c.wait()

        for g in range(NG):
            s = acc_v[pl.ds(g * L, L)]
            for j in range(N_SPARSE):
                s = s + emb_v[j, pl.ds(g * L, L)]
            acc_v[pl.ds(g * L, L)] = s

        pltpu.sync_copy(acc_v, out_hbm.at[pl.ds(base, RPW)])

    return k


def kernel(X, table, W, bias):
    wb = jnp.concatenate(
        [W[:, 0], bias, jnp.zeros((2,), jnp.float32)]
    )
    out = _build()(X.reshape(-1), table[:, 0], wb)
    return out.reshape(B, 1)


# speedup vs baseline: 4.4646x; 4.3506x over previous
import functools

import jax
import jax.numpy as jnp
from jax import lax
from jax.experimental import pallas as pl
from jax.experimental.pallas import tpu as pltpu
from jax.experimental.pallas import tpu_sc as plsc

B = 4096
N_SPARSE = 26
N_DENSE = 13
N_COLS = N_SPARSE + N_DENSE
VOCAB = 100000


@functools.cache
def _build():
    info = plsc.get_sparse_core_info()
    NC, NS, L = info.num_cores, info.num_subcores, info.num_lanes
    NW = NC * NS
    RPW = B // NW
    NG = RPW // L
    mesh = plsc.VectorSubcoreMesh(core_axis_name="c", subcore_axis_name="s")

    @functools.partial(
        pl.kernel,
        mesh=mesh,
        compiler_params=pltpu.CompilerParams(needs_layout_passes=False),
        out_type=jax.ShapeDtypeStruct((B,), jnp.float32),
        scratch_types=[
            pltpu.VMEM((RPW * N_COLS,), jnp.float32),
            pltpu.VMEM((N_SPARSE, RPW), jnp.int32),
            pltpu.VMEM((N_SPARSE, RPW), jnp.float32),
            pltpu.VMEM((RPW,), jnp.float32),
            pltpu.VMEM((L,), jnp.float32),
            pltpu.SemaphoreType.DMA,
        ],
    )
    def k(x_hbm, table_hbm, wb_hbm, out_hbm, x_v, idx_v, emb_v, acc_v, w_v, sem):
        wid = lax.axis_index("s") * NC + lax.axis_index("c")
        base = wid * RPW
        pltpu.sync_copy(x_hbm.at[pl.ds(base * N_COLS, RPW * N_COLS)], x_v)
        pltpu.sync_copy(wb_hbm, w_v)
        lanes = lax.iota(jnp.int32, L)

        for g in range(NG):
            flat = (lanes + (g * L)) * N_COLS
            for j in range(N_SPARSE):
                xf = plsc.load_gather(x_v, [flat + j])
                idx_v[j, pl.ds(g * L, L)] = xf.astype(jnp.int32) + j * VOCAB

        copies = [
            pltpu.async_copy(table_hbm.at[0].at[idx_v.at[j]], emb_v.at[j], sem)
            for j in range(N_SPARSE)
        ]

        wv = w_v[...]
        for g in range(NG):
            flat = (lanes + (g * L)) * N_COLS
            acc = jnp.full((L,), wv[N_DENSE], jnp.float32)
            for d in range(N_DENSE):
                xf = plsc.load_gather(x_v, [flat + N_SPARSE + d])
                acc = acc + xf * wv[d]
            acc_v[pl.ds(g * L, L)] = acc

        for c in copies:
            c.wait()

        for g in range(NG):
            s = acc_v[pl.ds(g * L, L)]
            for j in range(N_SPARSE):
                s = s + emb_v[j, pl.ds(g * L, L)]
            acc_v[pl.ds(g * L, L)] = s

        pltpu.sync_copy(acc_v, out_hbm.at[pl.ds(base, RPW)])

    return k


def kernel(X, table, W, bias):
    wb = jnp.concatenate(
        [W[:, 0], bias, jnp.zeros((2,), jnp.float32)]
    )
    out = _build()(X.reshape(-1), table.reshape(1, -1), wb)
    return out.reshape(B, 1)


# speedup vs baseline: 5.3302x; 1.1939x over previous
import functools

import jax
import jax.numpy as jnp
from jax import lax
from jax.experimental import pallas as pl
from jax.experimental.pallas import tpu as pltpu
from jax.experimental.pallas import tpu_sc as plsc

B = 4096
N_SPARSE = 26
N_DENSE = 13
N_COLS = N_SPARSE + N_DENSE
VOCAB = 100000


@functools.cache
def _build():
    info = plsc.get_sparse_core_info()
    NC, NS, L = info.num_cores, info.num_subcores, info.num_lanes
    NW = NC * NS
    RPW = B // NW
    NG = RPW // L
    mesh = plsc.VectorSubcoreMesh(core_axis_name="c", subcore_axis_name="s")

    @functools.partial(
        pl.kernel,
        mesh=mesh,
        compiler_params=pltpu.CompilerParams(needs_layout_passes=False),
        out_type=jax.ShapeDtypeStruct((B,), jnp.float32),
        scratch_types=[
            pltpu.VMEM((N_COLS, RPW), jnp.float32),
            pltpu.VMEM((N_SPARSE, RPW), jnp.int32),
            pltpu.VMEM((N_SPARSE, RPW), jnp.float32),
            pltpu.VMEM((RPW,), jnp.float32),
            pltpu.VMEM((L,), jnp.float32),
            pltpu.SemaphoreType.DMA,
        ],
    )
    def k(x_hbm, table_hbm, wb_hbm, out_hbm, x_v, idx_v, emb_v, acc_v, w_v, sem):
        wid = lax.axis_index("s") * NC + lax.axis_index("c")
        base = wid * RPW
        pltpu.sync_copy(x_hbm.at[:, pl.ds(base, RPW)], x_v)
        pltpu.sync_copy(wb_hbm, w_v)

        copies = []
        for j in range(N_SPARSE):
            for g in range(NG):
                xf = x_v[j, pl.ds(g * L, L)]
                idx_v[j, pl.ds(g * L, L)] = xf.astype(jnp.int32) + j * VOCAB
            copies.append(
                pltpu.async_copy(table_hbm.at[0].at[idx_v.at[j]], emb_v.at[j], sem)
            )

        wv = w_v[...]
        for g in range(NG):
            acc = jnp.full((L,), wv[N_DENSE], jnp.float32)
            for d in range(N_DENSE):
                acc = acc + x_v[N_SPARSE + d, pl.ds(g * L, L)] * wv[d]
            acc_v[pl.ds(g * L, L)] = acc

        for c in copies:
            c.wait()

        for g in range(NG):
            s = acc_v[pl.ds(g * L, L)]
            for j in range(N_SPARSE):
                s = s + emb_v[j, pl.ds(g * L, L)]
            acc_v[pl.ds(g * L, L)] = s

        pltpu.sync_copy(acc_v, out_hbm.at[pl.ds(base, RPW)])

    return k


def kernel(X, table, W, bias):
    wb = jnp.concatenate(
        [W[:, 0], bias, jnp.zeros((2,), jnp.float32)]
    )
    out = _build()(X.T, table.reshape(1, -1), wb)
    return out.reshape(B, 1)


# speedup vs baseline: 5.4207x; 1.0170x over previous
import functools

import jax
import jax.numpy as jnp
from jax import lax
from jax.experimental import pallas as pl
from jax.experimental.pallas import tpu as pltpu
from jax.experimental.pallas import tpu_sc as plsc

B = 4096
N_SPARSE = 26
N_DENSE = 13
N_COLS = N_SPARSE + N_DENSE
VOCAB = 100000


@functools.cache
def _build():
    info = plsc.get_sparse_core_info()
    NC, NS, L = info.num_cores, info.num_subcores, info.num_lanes
    NW = NC * NS
    RPW = B // NW
    NG = RPW // L
    mesh = plsc.VectorSubcoreMesh(core_axis_name="c", subcore_axis_name="s")

    @functools.partial(
        pl.kernel,
        mesh=mesh,
        compiler_params=pltpu.CompilerParams(needs_layout_passes=False),
        out_type=jax.ShapeDtypeStruct((B,), jnp.float32),
        scratch_types=[
            pltpu.VMEM((N_COLS, RPW), jnp.float32),
            pltpu.VMEM((N_SPARSE, RPW), jnp.int32),
            pltpu.VMEM((N_SPARSE, RPW), jnp.float32),
            pltpu.VMEM((RPW,), jnp.float32),
            pltpu.VMEM((N_DENSE,), jnp.float32),
            pltpu.VMEM((1,), jnp.float32),
            pltpu.SemaphoreType.DMA,
        ],
    )
    def k(x_hbm, table_hbm, w_hbm, b_hbm, out_hbm,
          x_v, idx_v, emb_v, acc_v, w_v, b_v, sem):
        wid = lax.axis_index("s") * NC + lax.axis_index("c")
        base = wid * RPW
        cx = pltpu.async_copy(x_hbm.at[:, pl.ds(base, RPW)], x_v, sem)
        cw = pltpu.async_copy(w_hbm, w_v, sem)
        cb = pltpu.async_copy(b_hbm, b_v, sem)
        cx.wait()
        cw.wait()
        cb.wait()
        lanes = lax.iota(jnp.int32, L)

        copies = []
        for j in range(N_SPARSE):
            for g in range(NG):
                xf = x_v[j, pl.ds(g * L, L)]
                idx_v[j, pl.ds(g * L, L)] = xf.astype(jnp.int32) + j * VOCAB
            copies.append(
                pltpu.async_copy(table_hbm.at[0].at[idx_v.at[j]], emb_v.at[j], sem)
            )

        wv = plsc.load_gather(w_v, [jnp.minimum(lanes, N_DENSE - 1)])
        bias_vec = plsc.load_gather(b_v, [jnp.zeros((L,), jnp.int32)])
        for g in range(NG):
            acc = bias_vec
            for d in range(N_DENSE):
                acc = acc + x_v[N_SPARSE + d, pl.ds(g * L, L)] * wv[d]
            acc_v[pl.ds(g * L, L)] = acc

        for c in copies:
            c.wait()

        for g in range(NG):
            s = acc_v[pl.ds(g * L, L)]
            for j in range(N_SPARSE):
                s = s + emb_v[j, pl.ds(g * L, L)]
            acc_v[pl.ds(g * L, L)] = s

        pltpu.sync_copy(acc_v, out_hbm.at[pl.ds(base, RPW)])

    return k


def kernel(X, table, W, bias):
    out = _build()(X.T, table.reshape(1, -1), W.reshape(-1), bias)
    return out.reshape(B, 1)


# speedup vs baseline: 5.5225x; 1.0188x over previous
import functools

import jax
import jax.numpy as jnp
from jax import lax
from jax.experimental import pallas as pl
from jax.experimental.pallas import tpu as pltpu
from jax.experimental.pallas import tpu_sc as plsc

B = 4096
N_SPARSE = 26
N_DENSE = 13
N_COLS = N_SPARSE + N_DENSE
VOCAB = 100000


@functools.cache
def _build():
    info = plsc.get_sparse_core_info()
    NC, NS, L = info.num_cores, info.num_subcores, info.num_lanes
    NW = NC * NS
    RPW = B // NW
    NG = RPW // L
    mesh = plsc.VectorSubcoreMesh(core_axis_name="c", subcore_axis_name="s")

    @functools.partial(
        pl.kernel,
        mesh=mesh,
        compiler_params=pltpu.CompilerParams(needs_layout_passes=False),
        out_type=jax.ShapeDtypeStruct((B,), jnp.float32),
        scratch_types=[
            pltpu.VMEM((N_COLS, RPW), jnp.float32),
            pltpu.VMEM((N_SPARSE * RPW,), jnp.int32),
            pltpu.VMEM((N_SPARSE * RPW,), jnp.float32),
            pltpu.VMEM((RPW,), jnp.float32),
            pltpu.VMEM((N_DENSE,), jnp.float32),
            pltpu.VMEM((1,), jnp.float32),
            pltpu.SemaphoreType.DMA,
        ],
    )
    def k(x_hbm, table_hbm, w_hbm, b_hbm, out_hbm,
          x_v, idx_v, emb_v, acc_v, w_v, b_v, sem):
        wid = lax.axis_index("s") * NC + lax.axis_index("c")
        base = wid * RPW
        cx = pltpu.async_copy(x_hbm.at[:, pl.ds(base, RPW)], x_v, sem)
        cw = pltpu.async_copy(w_hbm, w_v, sem)
        cb = pltpu.async_copy(b_hbm, b_v, sem)
        cx.wait()
        cw.wait()
        cb.wait()
        lanes = lax.iota(jnp.int32, L)

        for j in range(N_SPARSE):
            for g in range(NG):
                xf = x_v[j, pl.ds(g * L, L)]
                idx_v[pl.ds(j * RPW + g * L, L)] = xf.astype(jnp.int32) + j * VOCAB
        cg = pltpu.async_copy(table_hbm.at[0].at[idx_v], emb_v, sem)

        wv = plsc.load_gather(w_v, [jnp.minimum(lanes, N_DENSE - 1)])
        bias_vec = plsc.load_gather(b_v, [jnp.zeros((L,), jnp.int32)])
        for g in range(NG):
            acc = bias_vec
            for d in range(N_DENSE):
                acc = acc + x_v[N_SPARSE + d, pl.ds(g * L, L)] * wv[d]
            acc_v[pl.ds(g * L, L)] = acc

        cg.wait()

        for g in range(NG):
            s = acc_v[pl.ds(g * L, L)]
            for j in range(N_SPARSE):
                s = s + emb_v[pl.ds(j * RPW + g * L, L)]
            acc_v[pl.ds(g * L, L)] = s

        pltpu.sync_copy(acc_v, out_hbm.at[pl.ds(base, RPW)])

    return k


def kernel(X, table, W, bias):
    out = _build()(X.T, table.reshape(1, -1), W.reshape(-1), bias)
    return out.reshape(B, 1)


# speedup vs baseline: 5.7958x; 1.0495x over previous
import functools

import jax
import jax.numpy as jnp
from jax import lax
from jax.experimental import pallas as pl
from jax.experimental.pallas import tpu as pltpu
from jax.experimental.pallas import tpu_sc as plsc

B = 4096
N_SPARSE = 26
N_DENSE = 13
N_COLS = N_SPARSE + N_DENSE
VOCAB = 100000


@functools.cache
def _build():
    info = plsc.get_sparse_core_info()
    NC, NS, L = info.num_cores, info.num_subcores, info.num_lanes
    NW = NC * NS
    RPW = B // NW
    NG = RPW // L
    mesh = plsc.VectorSubcoreMesh(core_axis_name="c", subcore_axis_name="s")

    @functools.partial(
        pl.kernel,
        mesh=mesh,
        compiler_params=pltpu.CompilerParams(needs_layout_passes=False),
        out_type=jax.ShapeDtypeStruct((B,), jnp.float32),
        scratch_types=[
            pltpu.VMEM((N_COLS, RPW), jnp.float32),
            pltpu.VMEM((N_SPARSE * RPW // 2,), jnp.int32),
            pltpu.VMEM((N_SPARSE * RPW // 2,), jnp.int32),
            pltpu.VMEM((N_SPARSE * RPW // 2,), jnp.float32),
            pltpu.VMEM((N_SPARSE * RPW // 2,), jnp.float32),
            pltpu.VMEM((RPW,), jnp.float32),
            pltpu.VMEM((N_DENSE,), jnp.float32),
            pltpu.VMEM((1,), jnp.float32),
            pltpu.SemaphoreType.DMA,
            pltpu.SemaphoreType.DMA,
        ],
    )
    def k(x_hbm, table_hbm, w_hbm, b_hbm, out_hbm,
          x_v, idx_a, idx_b, emb_a, emb_b, acc_v, w_v, b_v, sem_a, sem_b):
        wid = lax.axis_index("s") * NC + lax.axis_index("c")
        base = wid * RPW
        HF = N_SPARSE // 2
        cx = pltpu.async_copy(x_hbm.at[:, pl.ds(base, RPW)], x_v, sem_a)
        cw = pltpu.async_copy(w_hbm, w_v, sem_b)
        cb = pltpu.async_copy(b_hbm, b_v, sem_b)
        cx.wait()
        cw.wait()
        cb.wait()
        lanes = lax.iota(jnp.int32, L)

        gathers = []
        for h, idx_v, emb_v, sem in ((0, idx_a, emb_a, sem_a),
                                     (1, idx_b, emb_b, sem_b)):
            for jj in range(HF):
                j = h * HF + jj
                for g in range(NG):
                    xf = x_v[j, pl.ds(g * L, L)]
                    idx_v[pl.ds(jj * RPW + g * L, L)] = (
                        xf.astype(jnp.int32) + j * VOCAB
                    )
            gathers.append(
                pltpu.async_copy(table_hbm.at[0].at[idx_v], emb_v, sem)
            )

        wv = plsc.load_gather(w_v, [jnp.minimum(lanes, N_DENSE - 1)])
        bias_vec = plsc.load_gather(b_v, [jnp.zeros((L,), jnp.int32)])
        for g in range(NG):
            acc = bias_vec
            for d in range(N_DENSE):
                acc = acc + x_v[N_SPARSE + d, pl.ds(g * L, L)] * wv[d]
            acc_v[pl.ds(g * L, L)] = acc

        for h, emb_v in ((0, emb_a), (1, emb_b)):
            gathers[h].wait()
            for g in range(NG):
                s = acc_v[pl.ds(g * L, L)]
                for jj in range(HF):
                    s = s + emb_v[pl.ds(jj * RPW + g * L, L)]
                acc_v[pl.ds(g * L, L)] = s

        pltpu.sync_copy(acc_v, out_hbm.at[pl.ds(base, RPW)])

    return k


def kernel(X, table, W, bias):
    out = _build()(X.T, table.reshape(1, -1), W.reshape(-1), bias)
    return out.reshape(B, 1)


# speedup vs baseline: 5.8524x; 1.0098x over previous
import functools

import jax
import jax.numpy as jnp
from jax import lax
from jax.experimental import pallas as pl
from jax.experimental.pallas import tpu as pltpu
from jax.experimental.pallas import tpu_sc as plsc

B = 4096
N_SPARSE = 26
N_DENSE = 13
N_COLS = N_SPARSE + N_DENSE
VOCAB = 100000


@functools.cache
def _build():
    info = plsc.get_sparse_core_info()
    NC, NS, L = info.num_cores, info.num_subcores, info.num_lanes
    NW = NC * NS
    RPW = B // NW
    NG = RPW // L
    mesh = plsc.VectorSubcoreMesh(core_axis_name="c", subcore_axis_name="s")

    @functools.partial(
        pl.kernel,
        mesh=mesh,
        compiler_params=pltpu.CompilerParams(needs_layout_passes=False),
        out_type=jax.ShapeDtypeStruct((B,), jnp.float32),
        scratch_types=[
            pltpu.VMEM((N_COLS, RPW), jnp.float32),
            pltpu.VMEM((N_SPARSE * RPW // 2,), jnp.int32),
            pltpu.VMEM((N_SPARSE * RPW // 2,), jnp.int32),
            pltpu.VMEM((N_SPARSE * RPW // 2,), jnp.float32),
            pltpu.VMEM((N_SPARSE * RPW // 2,), jnp.float32),
            pltpu.VMEM((RPW,), jnp.float32),
            pltpu.VMEM((N_DENSE,), jnp.float32),
            pltpu.VMEM((1,), jnp.float32),
            pltpu.SemaphoreType.DMA,
            pltpu.SemaphoreType.DMA,
        ],
    )
    def k(x_hbm, table_hbm, w_hbm, b_hbm, out_hbm,
          x_v, idx_a, idx_b, emb_a, emb_b, acc_v, w_v, b_v, sem_a, sem_b):
        wid = lax.axis_index("s") * NC + lax.axis_index("c")
        base = wid * RPW
        HF = N_SPARSE // 2
        cx1 = pltpu.async_copy(
            x_hbm.at[pl.ds(0, 16), pl.ds(base, RPW)], x_v.at[pl.ds(0, 16), :],
            sem_a)
        cx2 = pltpu.async_copy(
            x_hbm.at[pl.ds(16, 16), pl.ds(base, RPW)], x_v.at[pl.ds(16, 16), :],
            sem_b)
        cx3 = pltpu.async_copy(
            x_hbm.at[pl.ds(32, N_COLS - 32), pl.ds(base, RPW)],
            x_v.at[pl.ds(32, N_COLS - 32), :], sem_b)
        cw = pltpu.async_copy(w_hbm, w_v, sem_b)
        cb = pltpu.async_copy(b_hbm, b_v, sem_b)
        lanes = lax.iota(jnp.int32, L)

        gathers = []
        cx1.wait()
        for h, idx_v, emb_v, sem in ((0, idx_a, emb_a, sem_a),
                                     (1, idx_b, emb_b, sem_b)):
            if h == 1:
                cx2.wait()
                cx3.wait()
                cw.wait()
                cb.wait()
            for jj in range(HF):
                j = h * HF + jj
                for g in range(NG):
                    xf = x_v[j, pl.ds(g * L, L)]
                    idx_v[pl.ds(jj * RPW + g * L, L)] = (
                        xf.astype(jnp.int32) + j * VOCAB
                    )
            gathers.append(
                pltpu.async_copy(table_hbm.at[0].at[idx_v], emb_v, sem)
            )

        wv = plsc.load_gather(w_v, [jnp.minimum(lanes, N_DENSE - 1)])
        bias_vec = plsc.load_gather(b_v, [jnp.zeros((L,), jnp.int32)])
        for g in range(NG):
            acc = bias_vec
            for d in range(N_DENSE):
                acc = acc + x_v[N_SPARSE + d, pl.ds(g * L, L)] * wv[d]
            acc_v[pl.ds(g * L, L)] = acc

        for h, emb_v in ((0, emb_a), (1, emb_b)):
            gathers[h].wait()
            for g in range(NG):
                s = acc_v[pl.ds(g * L, L)]
                for jj in range(HF):
                    s = s + emb_v[pl.ds(jj * RPW + g * L, L)]
                acc_v[pl.ds(g * L, L)] = s

        pltpu.sync_copy(acc_v, out_hbm.at[pl.ds(base, RPW)])

    return k


def kernel(X, table, W, bias):
    out = _build()(X.T, table.reshape(1, -1), W.reshape(-1), bias)
    return out.reshape(B, 1)
